# baseline (speedup 1.0000x reference)
import jax
import jax.numpy as jnp
from jax.experimental import pallas as pl

_K = 32
_ROW_BLK = 256
_BISECT = 12
_FIX = 512


def _phase_a_body(xb_ref, xt_ref, th_ref, hi_ref, rc_ref, cnt_ref):
    i = pl.program_id(0)

    @pl.when(i == 0)
    def _init():
        cnt_ref[...] = jnp.zeros_like(cnt_ref)

    sim = jnp.dot(xb_ref[...], xt_ref[...], preferred_element_type=jnp.float32)
    n = sim.shape[1]

    m = sim[:, 0:128]
    for c in range(1, n // 128):
        m = jnp.maximum(m, sim[:, c * 128:(c + 1) * 128])

    v = m
    m1 = jnp.max(v, axis=1, keepdims=True)
    v = jnp.where(v == m1, -jnp.inf, v)
    hi = jnp.max(v, axis=1, keepdims=True)
    v = jnp.where(v == hi, -jnp.inf, v)
    for _ in range(_K - 3):
        mk = jnp.max(v, axis=1, keepdims=True)
        v = jnp.where(v == mk, -jnp.inf, v)
    lo = jnp.max(v, axis=1, keepdims=True)

    for _ in range(_BISECT):
        mid = 0.5 * (lo + hi)
        c = jnp.sum((sim >= mid).astype(jnp.float32), axis=1, keepdims=True)
        p = c >= _K
        lo = jnp.where(p, mid, lo)
        hi = jnp.where(p, hi, mid)

    mask = (sim >= lo).astype(jnp.float32)
    rc = jnp.sum(mask, axis=1, keepdims=True)
    th_ref[...] = jnp.broadcast_to(lo, th_ref.shape)
    hi_ref[...] = jnp.broadcast_to(hi, hi_ref.shape)
    rc_ref[...] = jnp.broadcast_to(rc, rc_ref.shape)
    cnt_ref[...] += jnp.sum(mask, axis=0, keepdims=True)


def _phase_b_body(xg_ref, xt_ref, old_ref, ohi_ref, tf_ref, dc_ref):
    i = pl.program_id(0)

    @pl.when(i == 0)
    def _init():
        dc_ref[...] = jnp.zeros_like(dc_ref)

    sim = jnp.dot(xg_ref[...], xt_ref[...], preferred_element_type=jnp.float32)
    lo_old = old_ref[...][:, :1]
    lo = lo_old
    hi = ohi_ref[...][:, :1]
    for _ in range(_BISECT + 6):
        mid = 0.5 * (lo + hi)
        c = jnp.sum((sim >= mid).astype(jnp.float32), axis=1, keepdims=True)
        p = c >= _K
        lo = jnp.where(p, mid, lo)
        hi = jnp.where(p, hi, mid)
    t = lo
    delta = (sim >= lo_old).astype(jnp.float32) - (sim >= t).astype(jnp.float32)
    tf_ref[...] = jnp.broadcast_to(t, tf_ref.shape)
    dc_ref[...] += jnp.sum(delta, axis=0, keepdims=True)


def _phase_c_body(xb_ref, xt_ref, thc_ref, thr_ref, rdc_ref, rdr_ref, adj_ref):
    s = jnp.dot(xb_ref[...], xt_ref[...], preferred_element_type=jnp.float32)
    ti = thc_ref[...][:, :1]
    tj = thr_ref[...]
    mi = (s >= ti).astype(jnp.float32)
    mj = (s >= tj).astype(jnp.float32)
    ri = rdc_ref[...][:, :1]
    rj = rdr_ref[...]
    adj_ref[...] = (mi + mj) * ((0.5 * ri) * rj)


def kernel(x, temperature):
    del temperature
    n, d = x.shape
    r = min(_ROW_BLK, n)
    nb = n // r
    xt = x.T
    f32 = jnp.float32

    th, hi, rc, cnt = pl.pallas_call(
        _phase_a_body,
        grid=(nb,),
        in_specs=[
            pl.BlockSpec((r, d), lambda i: (i, 0)),
            pl.BlockSpec((d, n), lambda i: (0, 0)),
        ],
        out_specs=[
            pl.BlockSpec((r, 128), lambda i: (i, 0)),
            pl.BlockSpec((r, 128), lambda i: (i, 0)),
            pl.BlockSpec((r, 128), lambda i: (i, 0)),
            pl.BlockSpec((1, n), lambda i: (0, 0)),
        ],
        out_shape=[
            jax.ShapeDtypeStruct((n, 128), f32),
            jax.ShapeDtypeStruct((n, 128), f32),
            jax.ShapeDtypeStruct((n, 128), f32),
            jax.ShapeDtypeStruct((1, n), f32),
        ],
    )(x, xt)

    nfix = min(_FIX, n)
    rfix = min(r, nfix)
    i32 = jnp.int32
    flag = rc[:, 0] > (_K + 0.5)
    ar = jnp.arange(n, dtype=i32)
    slot_f = jnp.where(flag, jnp.cumsum(flag.astype(i32)) - 1, nfix)
    flagged = jnp.full((nfix,), -1, i32).at[slot_f].set(ar, mode="drop")
    slot_p = jnp.where(~flag, jnp.cumsum((~flag).astype(i32)) - 1, nfix)
    padrow = jnp.zeros((nfix,), i32).at[slot_p].set(ar, mode="drop")
    fix_idx = jnp.where(flagged >= 0, flagged, padrow)
    xg = x[fix_idx]
    lo_old = th[fix_idx]
    hi_old = hi[fix_idx]

    tf, dc = pl.pallas_call(
        _phase_b_body,
        grid=(nfix // rfix,),
        in_specs=[
            pl.BlockSpec((rfix, d), lambda i: (i, 0)),
            pl.BlockSpec((d, n), lambda i: (0, 0)),
            pl.BlockSpec((rfix, 128), lambda i: (i, 0)),
            pl.BlockSpec((rfix, 128), lambda i: (i, 0)),
        ],
        out_specs=[
            pl.BlockSpec((rfix, 128), lambda i: (i, 0)),
            pl.BlockSpec((1, n), lambda i: (0, 0)),
        ],
        out_shape=[
            jax.ShapeDtypeStruct((nfix, 128), f32),
            jax.ShapeDtypeStruct((1, n), f32),
        ],
    )(xg, xt, lo_old, hi_old)

    th_v = th[:, 0].at[fix_idx].set(tf[:, 0])
    cnt_v = cnt[0] - dc[0]
    rdeg = jax.lax.rsqrt(0.5 * (jnp.float32(_K) + cnt_v))
    thc = jnp.broadcast_to(th_v[:, None], (n, 128))
    thr = th_v.reshape(1, n)
    rdc = jnp.broadcast_to(rdeg[:, None], (n, 128))
    rdr = rdeg.reshape(1, n)

    adj = pl.pallas_call(
        _phase_c_body,
        grid=(nb,),
        in_specs=[
            pl.BlockSpec((r, d), lambda i: (i, 0)),
            pl.BlockSpec((d, n), lambda i: (0, 0)),
            pl.BlockSpec((r, 128), lambda i: (i, 0)),
            pl.BlockSpec((1, n), lambda i: (0, 0)),
            pl.BlockSpec((r, 128), lambda i: (i, 0)),
            pl.BlockSpec((1, n), lambda i: (0, 0)),
        ],
        out_specs=pl.BlockSpec((r, n), lambda i: (i, 0)),
        out_shape=jax.ShapeDtypeStruct((n, n), f32),
    )(x, xt, thc, thr, rdc, rdr)
    return adj


# speedup vs baseline: 1.1239x; 1.1239x over previous
import jax
import jax.numpy as jnp
from jax.experimental import pallas as pl

_K = 32
_ROW_BLK = 256
_BISECT = 12
_FIX = 512


def _phase_a_body(xb_ref, xt_ref, th_ref, hi_ref, rc_ref, cnt_ref):
    i = pl.program_id(0)

    @pl.when(i == 0)
    def _init():
        cnt_ref[...] = jnp.zeros_like(cnt_ref)

    sim = jnp.dot(xb_ref[...], xt_ref[...], preferred_element_type=jnp.float32)
    n = sim.shape[1]

    m = sim[:, 0:128]
    for c in range(1, n // 128):
        m = jnp.maximum(m, sim[:, c * 128:(c + 1) * 128])

    v = m
    m1 = jnp.max(v, axis=1, keepdims=True)
    v = jnp.where(v == m1, -jnp.inf, v)
    hi = jnp.max(v, axis=1, keepdims=True)
    v = jnp.where(v == hi, -jnp.inf, v)
    for _ in range(_K - 3):
        mk = jnp.max(v, axis=1, keepdims=True)
        v = jnp.where(v == mk, -jnp.inf, v)
    lo = jnp.max(v, axis=1, keepdims=True)

    for _ in range(_BISECT):
        mid = 0.5 * (lo + hi)
        c = jnp.sum((sim >= mid).astype(jnp.float32), axis=1, keepdims=True)
        p = c >= _K
        lo = jnp.where(p, mid, lo)
        hi = jnp.where(p, hi, mid)

    mask = (sim >= lo).astype(jnp.float32)
    rc = jnp.sum(mask, axis=1, keepdims=True)
    th_ref[...] = jnp.broadcast_to(lo, th_ref.shape)
    hi_ref[...] = jnp.broadcast_to(hi, hi_ref.shape)
    rc_ref[...] = jnp.broadcast_to(rc, rc_ref.shape)
    cnt_ref[...] += jnp.sum(mask, axis=0, keepdims=True)


def _phase_b_body(xg_ref, xt_ref, old_ref, ohi_ref, tf_ref, dc_ref):
    i = pl.program_id(0)

    @pl.when(i == 0)
    def _init():
        dc_ref[...] = jnp.zeros_like(dc_ref)

    sim = jnp.dot(xg_ref[...], xt_ref[...], preferred_element_type=jnp.float32)
    lo_old = old_ref[...][:, :1]
    lo = lo_old
    hi = ohi_ref[...][:, :1]
    for _ in range(_BISECT + 6):
        mid = 0.5 * (lo + hi)
        c = jnp.sum((sim >= mid).astype(jnp.float32), axis=1, keepdims=True)
        p = c >= _K
        lo = jnp.where(p, mid, lo)
        hi = jnp.where(p, hi, mid)
    t = lo
    delta = (sim >= lo_old).astype(jnp.float32) - (sim >= t).astype(jnp.float32)
    tf_ref[...] = jnp.broadcast_to(t, tf_ref.shape)
    dc_ref[...] += jnp.sum(delta, axis=0, keepdims=True)


def _phase_c_body(xb_ref, xt_ref, thc_ref, thr_ref, rdc_ref, rdr_ref, adj_ref):
    s = jnp.dot(xb_ref[...], xt_ref[...], preferred_element_type=jnp.float32)
    ti = thc_ref[...][:, :1]
    tj = thr_ref[...]
    mi = (s >= ti).astype(jnp.float32)
    mj = (s >= tj).astype(jnp.float32)
    ri = rdc_ref[...][:, :1]
    rj = rdr_ref[...]
    adj_ref[...] = (mi + mj) * ((0.5 * ri) * rj)


def kernel(x, temperature):
    del temperature
    n, d = x.shape
    r = min(_ROW_BLK, n)
    nb = n // r
    xt = x.T
    f32 = jnp.float32

    th, hi, rc, cnt = pl.pallas_call(
        _phase_a_body,
        grid=(nb,),
        in_specs=[
            pl.BlockSpec((r, d), lambda i: (i, 0)),
            pl.BlockSpec((d, n), lambda i: (0, 0)),
        ],
        out_specs=[
            pl.BlockSpec((r, 128), lambda i: (i, 0)),
            pl.BlockSpec((r, 128), lambda i: (i, 0)),
            pl.BlockSpec((r, 128), lambda i: (i, 0)),
            pl.BlockSpec((1, n), lambda i: (0, 0)),
        ],
        out_shape=[
            jax.ShapeDtypeStruct((n, 128), f32),
            jax.ShapeDtypeStruct((n, 128), f32),
            jax.ShapeDtypeStruct((n, 128), f32),
            jax.ShapeDtypeStruct((1, n), f32),
        ],
    )(x, xt)

    nfix = min(_FIX, n)
    rfix = min(r, nfix)
    _, fix_idx = jax.lax.top_k(rc[:, 0], nfix)
    xg = x[fix_idx]
    lo_old = th[fix_idx]
    hi_old = hi[fix_idx]

    tf, dc = pl.pallas_call(
        _phase_b_body,
        grid=(nfix // rfix,),
        in_specs=[
            pl.BlockSpec((rfix, d), lambda i: (i, 0)),
            pl.BlockSpec((d, n), lambda i: (0, 0)),
            pl.BlockSpec((rfix, 128), lambda i: (i, 0)),
            pl.BlockSpec((rfix, 128), lambda i: (i, 0)),
        ],
        out_specs=[
            pl.BlockSpec((rfix, 128), lambda i: (i, 0)),
            pl.BlockSpec((1, n), lambda i: (0, 0)),
        ],
        out_shape=[
            jax.ShapeDtypeStruct((nfix, 128), f32),
            jax.ShapeDtypeStruct((1, n), f32),
        ],
    )(xg, xt, lo_old, hi_old)

    th_v = th[:, 0].at[fix_idx].set(tf[:, 0])
    cnt_v = cnt[0] - dc[0]
    rdeg = jax.lax.rsqrt(0.5 * (jnp.float32(_K) + cnt_v))
    thc = jnp.broadcast_to(th_v[:, None], (n, 128))
    thr = th_v.reshape(1, n)
    rdc = jnp.broadcast_to(rdeg[:, None], (n, 128))
    rdr = rdeg.reshape(1, n)

    adj = pl.pallas_call(
        _phase_c_body,
        grid=(nb,),
        in_specs=[
            pl.BlockSpec((r, d), lambda i: (i, 0)),
            pl.BlockSpec((d, n), lambda i: (0, 0)),
            pl.BlockSpec((r, 128), lambda i: (i, 0)),
            pl.BlockSpec((1, n), lambda i: (0, 0)),
            pl.BlockSpec((r, 128), lambda i: (i, 0)),
            pl.BlockSpec((1, n), lambda i: (0, 0)),
        ],
        out_specs=pl.BlockSpec((r, n), lambda i: (i, 0)),
        out_shape=jax.ShapeDtypeStruct((n, n), f32),
    )(x, xt, thc, thr, rdc, rdr)
    return adj


# speedup vs baseline: 1.1426x; 1.0167x over previous
import jax
import jax.numpy as jnp
from jax.experimental import pallas as pl

_K = 32
_ROW_BLK = 256
_BISECT = 12
_FIX = 512


def _phase_a_body(xb_ref, xt_ref, th_ref, hi_ref, rc_ref, cnt_ref):
    i = pl.program_id(0)

    @pl.when(i == 0)
    def _init():
        cnt_ref[...] = jnp.zeros_like(cnt_ref)

    sim = jnp.dot(xb_ref[...], xt_ref[...], preferred_element_type=jnp.float32)
    n = sim.shape[1]

    m = sim[:, 0:128]
    for c in range(1, n // 128):
        m = jnp.maximum(m, sim[:, c * 128:(c + 1) * 128])

    v = m
    m1 = jnp.max(v, axis=1, keepdims=True)
    v = jnp.where(v == m1, -jnp.inf, v)
    hi = jnp.max(v, axis=1, keepdims=True)
    v = jnp.where(v == hi, -jnp.inf, v)
    for _ in range(_K - 3):
        mk = jnp.max(v, axis=1, keepdims=True)
        v = jnp.where(v == mk, -jnp.inf, v)
    lo = jnp.max(v, axis=1, keepdims=True)

    for _ in range(_BISECT):
        mid = 0.5 * (lo + hi)
        c = jnp.sum((sim >= mid).astype(jnp.float32), axis=1, keepdims=True)
        p = c >= _K
        lo = jnp.where(p, mid, lo)
        hi = jnp.where(p, hi, mid)

    mask = (sim >= lo).astype(jnp.float32)
    rc = jnp.sum(mask, axis=1, keepdims=True)
    th_ref[...] = jnp.broadcast_to(lo, th_ref.shape)
    hi_ref[...] = jnp.broadcast_to(hi, hi_ref.shape)
    rc_ref[...] = jnp.broadcast_to(rc, rc_ref.shape)
    cnt_ref[...] += jnp.sum(mask, axis=0, keepdims=True)


def _phase_b_body(xg_ref, xt_ref, old_ref, ohi_ref, tf_ref, dc_ref):
    i = pl.program_id(0)

    @pl.when(i == 0)
    def _init():
        dc_ref[...] = jnp.zeros_like(dc_ref)

    sim = jnp.dot(xg_ref[...], xt_ref[...], preferred_element_type=jnp.float32)
    lo_old = old_ref[...][:, :1]
    lo = lo_old
    hi = ohi_ref[...][:, :1]
    for _ in range(_BISECT + 6):
        mid = 0.5 * (lo + hi)
        c = jnp.sum((sim >= mid).astype(jnp.float32), axis=1, keepdims=True)
        p = c >= _K
        lo = jnp.where(p, mid, lo)
        hi = jnp.where(p, hi, mid)
    t = lo
    delta = (sim >= lo_old).astype(jnp.float32) - (sim >= t).astype(jnp.float32)
    tf_ref[...] = jnp.broadcast_to(t, tf_ref.shape)
    dc_ref[...] += jnp.sum(delta, axis=0, keepdims=True)


def _phase_c_body(xb_ref, xt_ref, tv3_ref, thr_ref, cv3_ref, cvr_ref, adj_ref):
    s = jnp.dot(xb_ref[...], xt_ref[...], preferred_element_type=jnp.float32)
    rr = s.shape[0]
    half_k = jnp.float32(0.5 * _K)
    tj = thr_ref[...]
    rj = jax.lax.rsqrt(half_k + 0.5 * cvr_ref[...])
    eye = (jax.lax.broadcasted_iota(jnp.int32, (rr, rr), 0)
           == jax.lax.broadcasted_iota(jnp.int32, (rr, rr), 1)
           ).astype(jnp.float32)
    ti = jnp.sum(eye * tv3_ref[0], axis=1, keepdims=True)
    ci = jnp.sum(eye * cv3_ref[0], axis=1, keepdims=True)
    ri = jax.lax.rsqrt(half_k + 0.5 * ci)
    mi = (s >= ti).astype(jnp.float32)
    mj = (s >= tj).astype(jnp.float32)
    adj_ref[...] = (mi + mj) * ((0.5 * ri) * rj)


def kernel(x, temperature):
    del temperature
    n, d = x.shape
    r = min(_ROW_BLK, n)
    nb = n // r
    xt = x.T
    f32 = jnp.float32

    th, hi, rc, cnt = pl.pallas_call(
        _phase_a_body,
        grid=(nb,),
        in_specs=[
            pl.BlockSpec((r, d), lambda i: (i, 0)),
            pl.BlockSpec((d, n), lambda i: (0, 0)),
        ],
        out_specs=[
            pl.BlockSpec((r, 128), lambda i: (i, 0)),
            pl.BlockSpec((r, 128), lambda i: (i, 0)),
            pl.BlockSpec((r, 128), lambda i: (i, 0)),
            pl.BlockSpec((1, n), lambda i: (0, 0)),
        ],
        out_shape=[
            jax.ShapeDtypeStruct((n, 128), f32),
            jax.ShapeDtypeStruct((n, 128), f32),
            jax.ShapeDtypeStruct((n, 128), f32),
            jax.ShapeDtypeStruct((1, n), f32),
        ],
    )(x, xt)

    nfix = min(_FIX, n)
    rfix = min(r, nfix)
    _, fix_idx = jax.lax.top_k(jnp.abs(rc[:, 0] - _K), nfix)
    xg = x[fix_idx]
    lo_old = th[fix_idx]
    hi_old = hi[fix_idx]

    tf, dc = pl.pallas_call(
        _phase_b_body,
        grid=(nfix // rfix,),
        in_specs=[
            pl.BlockSpec((rfix, d), lambda i: (i, 0)),
            pl.BlockSpec((d, n), lambda i: (0, 0)),
            pl.BlockSpec((rfix, 128), lambda i: (i, 0)),
            pl.BlockSpec((rfix, 128), lambda i: (i, 0)),
        ],
        out_specs=[
            pl.BlockSpec((rfix, 128), lambda i: (i, 0)),
            pl.BlockSpec((1, n), lambda i: (0, 0)),
        ],
        out_shape=[
            jax.ShapeDtypeStruct((nfix, 128), f32),
            jax.ShapeDtypeStruct((1, n), f32),
        ],
    )(xg, xt, lo_old, hi_old)

    th_v = th[:, 0].at[fix_idx].set(tf[:, 0])
    cnt_v = cnt[0] - dc[0]
    tv3 = th_v.reshape(nb, 1, r)
    thr = th_v.reshape(1, n)
    cv3 = cnt_v.reshape(nb, 1, r)
    cvr = cnt_v.reshape(1, n)

    adj = pl.pallas_call(
        _phase_c_body,
        grid=(nb,),
        in_specs=[
            pl.BlockSpec((r, d), lambda i: (i, 0)),
            pl.BlockSpec((d, n), lambda i: (0, 0)),
            pl.BlockSpec((1, 1, r), lambda i: (i, 0, 0)),
            pl.BlockSpec((1, n), lambda i: (0, 0)),
            pl.BlockSpec((1, 1, r), lambda i: (i, 0, 0)),
            pl.BlockSpec((1, n), lambda i: (0, 0)),
        ],
        out_specs=pl.BlockSpec((r, n), lambda i: (i, 0)),
        out_shape=jax.ShapeDtypeStruct((n, n), f32),
    )(x, xt, tv3, thr, cv3, cvr)
    return adj


# speedup vs baseline: 1.2662x; 1.1082x over previous
import jax
import jax.numpy as jnp
from jax.experimental import pallas as pl

_K = 32
_ROW_BLK = 512
_BISECT = 12
_FIX = 512


def _phase_a_body(xb_ref, xt_ref, th_ref, hi_ref, rc_ref, cnt_ref):
    i = pl.program_id(0)

    @pl.when(i == 0)
    def _init():
        cnt_ref[...] = jnp.zeros_like(cnt_ref)

    sim = jnp.dot(xb_ref[...], xt_ref[...], preferred_element_type=jnp.float32)
    n = sim.shape[1]

    m = sim[:, 0:128]
    for c in range(1, n // 128):
        m = jnp.maximum(m, sim[:, c * 128:(c + 1) * 128])

    v = m
    m1 = jnp.max(v, axis=1, keepdims=True)
    v = jnp.where(v == m1, -jnp.inf, v)
    hi = jnp.max(v, axis=1, keepdims=True)
    v = jnp.where(v == hi, -jnp.inf, v)
    for _ in range(_K - 3):
        mk = jnp.max(v, axis=1, keepdims=True)
        v = jnp.where(v == mk, -jnp.inf, v)
    lo = jnp.max(v, axis=1, keepdims=True)

    for _ in range(_BISECT):
        mid = 0.5 * (lo + hi)
        c = jnp.sum((sim >= mid).astype(jnp.float32), axis=1, keepdims=True)
        p = c >= _K
        lo = jnp.where(p, mid, lo)
        hi = jnp.where(p, hi, mid)

    mask = (sim >= lo).astype(jnp.float32)
    rc = jnp.sum(mask, axis=1, keepdims=True)
    th_ref[...] = jnp.broadcast_to(lo, th_ref.shape)
    hi_ref[...] = jnp.broadcast_to(hi, hi_ref.shape)
    rc_ref[...] = jnp.broadcast_to(rc, rc_ref.shape)
    cnt_ref[...] += jnp.sum(mask, axis=0, keepdims=True)


def _phase_b_body(xg_ref, xt_ref, old_ref, ohi_ref, tf_ref, dc_ref):
    i = pl.program_id(0)

    @pl.when(i == 0)
    def _init():
        dc_ref[...] = jnp.zeros_like(dc_ref)

    sim = jnp.dot(xg_ref[...], xt_ref[...], preferred_element_type=jnp.float32)
    lo_old = old_ref[...][:, :1]
    lo = lo_old
    hi = ohi_ref[...][:, :1]
    for _ in range(_BISECT + 6):
        mid = 0.5 * (lo + hi)
        c = jnp.sum((sim >= mid).astype(jnp.float32), axis=1, keepdims=True)
        p = c >= _K
        lo = jnp.where(p, mid, lo)
        hi = jnp.where(p, hi, mid)
    t = lo
    delta = (sim >= lo_old).astype(jnp.float32) - (sim >= t).astype(jnp.float32)
    tf_ref[...] = jnp.broadcast_to(t, tf_ref.shape)
    dc_ref[...] += jnp.sum(delta, axis=0, keepdims=True)


def _phase_c_body(xb_ref, xt_ref, tv3_ref, thr_ref, cv3_ref, cvr_ref, adj_ref):
    s = jnp.dot(xb_ref[...], xt_ref[...], preferred_element_type=jnp.float32)
    rr = s.shape[0]
    half_k = jnp.float32(0.5 * _K)
    tj = thr_ref[...]
    rj = jax.lax.rsqrt(half_k + 0.5 * cvr_ref[...])
    eye = (jax.lax.broadcasted_iota(jnp.int32, (rr, rr), 0)
           == jax.lax.broadcasted_iota(jnp.int32, (rr, rr), 1)
           ).astype(jnp.float32)
    ti = jnp.sum(eye * tv3_ref[0], axis=1, keepdims=True)
    ci = jnp.sum(eye * cv3_ref[0], axis=1, keepdims=True)
    ri = jax.lax.rsqrt(half_k + 0.5 * ci)
    mi = (s >= ti).astype(jnp.float32)
    mj = (s >= tj).astype(jnp.float32)
    adj_ref[...] = (mi + mj) * ((0.5 * ri) * rj)


def kernel(x, temperature):
    del temperature
    n, d = x.shape
    r = min(_ROW_BLK, n)
    nb = n // r
    xt = x.T
    f32 = jnp.float32

    th, hi, rc, cnt = pl.pallas_call(
        _phase_a_body,
        grid=(nb,),
        in_specs=[
            pl.BlockSpec((r, d), lambda i: (i, 0)),
            pl.BlockSpec((d, n), lambda i: (0, 0)),
        ],
        out_specs=[
            pl.BlockSpec((r, 128), lambda i: (i, 0)),
            pl.BlockSpec((r, 128), lambda i: (i, 0)),
            pl.BlockSpec((r, 128), lambda i: (i, 0)),
            pl.BlockSpec((1, n), lambda i: (0, 0)),
        ],
        out_shape=[
            jax.ShapeDtypeStruct((n, 128), f32),
            jax.ShapeDtypeStruct((n, 128), f32),
            jax.ShapeDtypeStruct((n, 128), f32),
            jax.ShapeDtypeStruct((1, n), f32),
        ],
    )(x, xt)

    nfix = min(_FIX, n)
    rfix = min(r, nfix)
    _, fix_idx = jax.lax.top_k(jnp.abs(rc[:, 0] - _K), nfix)
    xg = x[fix_idx]
    lo_old = th[fix_idx]
    hi_old = hi[fix_idx]

    tf, dc = pl.pallas_call(
        _phase_b_body,
        grid=(nfix // rfix,),
        in_specs=[
            pl.BlockSpec((rfix, d), lambda i: (i, 0)),
            pl.BlockSpec((d, n), lambda i: (0, 0)),
            pl.BlockSpec((rfix, 128), lambda i: (i, 0)),
            pl.BlockSpec((rfix, 128), lambda i: (i, 0)),
        ],
        out_specs=[
            pl.BlockSpec((rfix, 128), lambda i: (i, 0)),
            pl.BlockSpec((1, n), lambda i: (0, 0)),
        ],
        out_shape=[
            jax.ShapeDtypeStruct((nfix, 128), f32),
            jax.ShapeDtypeStruct((1, n), f32),
        ],
    )(xg, xt, lo_old, hi_old)

    th_v = th[:, 0].at[fix_idx].set(tf[:, 0])
    cnt_v = cnt[0] - dc[0]
    tv3 = th_v.reshape(nb, 1, r)
    thr = th_v.reshape(1, n)
    cv3 = cnt_v.reshape(nb, 1, r)
    cvr = cnt_v.reshape(1, n)

    adj = pl.pallas_call(
        _phase_c_body,
        grid=(nb,),
        in_specs=[
            pl.BlockSpec((r, d), lambda i: (i, 0)),
            pl.BlockSpec((d, n), lambda i: (0, 0)),
            pl.BlockSpec((1, 1, r), lambda i: (i, 0, 0)),
            pl.BlockSpec((1, n), lambda i: (0, 0)),
            pl.BlockSpec((1, 1, r), lambda i: (i, 0, 0)),
            pl.BlockSpec((1, n), lambda i: (0, 0)),
        ],
        out_specs=pl.BlockSpec((r, n), lambda i: (i, 0)),
        out_shape=jax.ShapeDtypeStruct((n, n), f32),
    )(x, xt, tv3, thr, cv3, cvr)
    return adj


# speedup vs baseline: 1.3112x; 1.0356x over previous
import jax
import jax.numpy as jnp
from jax.experimental import pallas as pl

_K = 32
_ROW_BLK = 1024
_BISECT = 12
_FIX = 512


def _phase_a_body(xb_ref, xt_ref, th_ref, hi_ref, rc_ref, cnt_ref):
    i = pl.program_id(0)

    @pl.when(i == 0)
    def _init():
        cnt_ref[...] = jnp.zeros_like(cnt_ref)

    sim = jnp.dot(xb_ref[...], xt_ref[...], preferred_element_type=jnp.float32)
    n = sim.shape[1]

    m = sim[:, 0:128]
    for c in range(1, n // 128):
        m = jnp.maximum(m, sim[:, c * 128:(c + 1) * 128])

    v = m
    m1 = jnp.max(v, axis=1, keepdims=True)
    v = jnp.where(v == m1, -jnp.inf, v)
    hi = jnp.max(v, axis=1, keepdims=True)
    v = jnp.where(v == hi, -jnp.inf, v)
    for _ in range(_K - 3):
        mk = jnp.max(v, axis=1, keepdims=True)
        v = jnp.where(v == mk, -jnp.inf, v)
    lo = jnp.max(v, axis=1, keepdims=True)

    for _ in range(_BISECT):
        mid = 0.5 * (lo + hi)
        c = jnp.sum((sim >= mid).astype(jnp.float32), axis=1, keepdims=True)
        p = c >= _K
        lo = jnp.where(p, mid, lo)
        hi = jnp.where(p, hi, mid)

    mask = (sim >= lo).astype(jnp.float32)
    rc = jnp.sum(mask, axis=1, keepdims=True)
    th_ref[...] = jnp.broadcast_to(lo, th_ref.shape)
    hi_ref[...] = jnp.broadcast_to(hi, hi_ref.shape)
    rc_ref[...] = jnp.broadcast_to(rc, rc_ref.shape)
    cnt_ref[...] += jnp.sum(mask, axis=0, keepdims=True)


def _phase_b_body(xg_ref, xt_ref, old_ref, ohi_ref, tf_ref, dc_ref):
    i = pl.program_id(0)

    @pl.when(i == 0)
    def _init():
        dc_ref[...] = jnp.zeros_like(dc_ref)

    sim = jnp.dot(xg_ref[...], xt_ref[...], preferred_element_type=jnp.float32)
    lo_old = old_ref[...][:, :1]
    lo = lo_old
    hi = ohi_ref[...][:, :1]
    for _ in range(_BISECT + 6):
        mid = 0.5 * (lo + hi)
        c = jnp.sum((sim >= mid).astype(jnp.float32), axis=1, keepdims=True)
        p = c >= _K
        lo = jnp.where(p, mid, lo)
        hi = jnp.where(p, hi, mid)
    t = lo
    delta = (sim >= lo_old).astype(jnp.float32) - (sim >= t).astype(jnp.float32)
    tf_ref[...] = jnp.broadcast_to(t, tf_ref.shape)
    dc_ref[...] += jnp.sum(delta, axis=0, keepdims=True)


def _phase_c_body(xb_ref, xt_ref, tv3_ref, thr_ref, cv3_ref, cvr_ref, adj_ref):
    s = jnp.dot(xb_ref[...], xt_ref[...], preferred_element_type=jnp.float32)
    rr = s.shape[0]
    half_k = jnp.float32(0.5 * _K)
    tj = thr_ref[...]
    rj = jax.lax.rsqrt(half_k + 0.5 * cvr_ref[...])
    eye = (jax.lax.broadcasted_iota(jnp.int32, (rr, rr), 0)
           == jax.lax.broadcasted_iota(jnp.int32, (rr, rr), 1)
           ).astype(jnp.float32)
    ti = jnp.sum(eye * tv3_ref[0], axis=1, keepdims=True)
    ci = jnp.sum(eye * cv3_ref[0], axis=1, keepdims=True)
    ri = jax.lax.rsqrt(half_k + 0.5 * ci)
    mi = (s >= ti).astype(jnp.float32)
    mj = (s >= tj).astype(jnp.float32)
    adj_ref[...] = (mi + mj) * ((0.5 * ri) * rj)


def kernel(x, temperature):
    del temperature
    n, d = x.shape
    r = min(_ROW_BLK, n)
    nb = n // r
    xt = x.T
    f32 = jnp.float32

    th, hi, rc, cnt = pl.pallas_call(
        _phase_a_body,
        grid=(nb,),
        in_specs=[
            pl.BlockSpec((r, d), lambda i: (i, 0)),
            pl.BlockSpec((d, n), lambda i: (0, 0)),
        ],
        out_specs=[
            pl.BlockSpec((r, 128), lambda i: (i, 0)),
            pl.BlockSpec((r, 128), lambda i: (i, 0)),
            pl.BlockSpec((r, 128), lambda i: (i, 0)),
            pl.BlockSpec((1, n), lambda i: (0, 0)),
        ],
        out_shape=[
            jax.ShapeDtypeStruct((n, 128), f32),
            jax.ShapeDtypeStruct((n, 128), f32),
            jax.ShapeDtypeStruct((n, 128), f32),
            jax.ShapeDtypeStruct((1, n), f32),
        ],
    )(x, xt)

    nfix = min(_FIX, n)
    rfix = min(r, nfix)
    _, fix_idx = jax.lax.top_k(jnp.abs(rc[:, 0] - _K), nfix)
    xg = x[fix_idx]
    lo_old = th[fix_idx]
    hi_old = hi[fix_idx]

    tf, dc = pl.pallas_call(
        _phase_b_body,
        grid=(nfix // rfix,),
        in_specs=[
            pl.BlockSpec((rfix, d), lambda i: (i, 0)),
            pl.BlockSpec((d, n), lambda i: (0, 0)),
            pl.BlockSpec((rfix, 128), lambda i: (i, 0)),
            pl.BlockSpec((rfix, 128), lambda i: (i, 0)),
        ],
        out_specs=[
            pl.BlockSpec((rfix, 128), lambda i: (i, 0)),
            pl.BlockSpec((1, n), lambda i: (0, 0)),
        ],
        out_shape=[
            jax.ShapeDtypeStruct((nfix, 128), f32),
            jax.ShapeDtypeStruct((1, n), f32),
        ],
    )(xg, xt, lo_old, hi_old)

    th_v = th[:, 0].at[fix_idx].set(tf[:, 0])
    cnt_v = cnt[0] - dc[0]
    tv3 = th_v.reshape(nb, 1, r)
    thr = th_v.reshape(1, n)
    cv3 = cnt_v.reshape(nb, 1, r)
    cvr = cnt_v.reshape(1, n)

    adj = pl.pallas_call(
        _phase_c_body,
        grid=(nb,),
        in_specs=[
            pl.BlockSpec((r, d), lambda i: (i, 0)),
            pl.BlockSpec((d, n), lambda i: (0, 0)),
            pl.BlockSpec((1, 1, r), lambda i: (i, 0, 0)),
            pl.BlockSpec((1, n), lambda i: (0, 0)),
            pl.BlockSpec((1, 1, r), lambda i: (i, 0, 0)),
            pl.BlockSpec((1, n), lambda i: (0, 0)),
        ],
        out_specs=pl.BlockSpec((r, n), lambda i: (i, 0)),
        out_shape=jax.ShapeDtypeStruct((n, n), f32),
    )(x, xt, tv3, thr, cv3, cvr)
    return adj
